# baseline (speedup 1.0000x reference)
import functools

import jax
import jax.numpy as jnp
from jax import lax
from jax.experimental import pallas as pl
from jax.experimental.pallas import tpu as pltpu
from jax.experimental.pallas import tpu_sc as plsc

_CHUNK = 4096


def _codes_body(zm2_ref, et_ref, a_ref, c_ref, codes_ref):
    zm2 = zm2_ref[...]
    et = et_ref[...]
    k = et.shape[1]
    bm = zm2.shape[0]
    a = a_ref[...]
    c = c_ref[...]
    n_chunk = k // _CHUNK
    cols = _CHUNK // 128
    wv, wi = [], []
    for t in range(n_chunk):
        acc_v = None
        acc_j = None
        for j in range(cols):
            lo = t * _CHUNK + j * 128
            m2 = jnp.dot(zm2, et[:, lo:lo + 128],
                         preferred_element_type=jnp.float32)
            v = (a + m2) + c[:, lo:lo + 128]
            if acc_v is None:
                acc_v = v
                acc_j = jnp.zeros((bm, 128), jnp.int32)
            else:
                lt = v < acc_v
                acc_v = jnp.where(lt, v, acc_v)
                acc_j = jnp.where(lt, j, acc_j)
        lane = lax.broadcasted_iota(jnp.int32, (bm, 128), 1)
        gk = t * _CHUNK + acc_j * 128 + lane
        mn = jnp.min(acc_v, axis=1, keepdims=True)
        idx = jnp.min(jnp.where(acc_v == mn, gk, k), axis=1)
        wv.append(mn[:, 0])
        wi.append(idx)

    def round_bf16(v):
        u = lax.bitcast_convert_type(v, jnp.uint32)
        lsb = (u >> jnp.uint32(16)) & jnp.uint32(1)
        u = (u + jnp.uint32(0x7FFF) + lsb) & jnp.uint32(0xFFFF0000)
        return lax.bitcast_convert_type(u, jnp.float32)

    def comb(av, ai, bv, bi):
        ab = round_bf16(av)
        take_b = (bv < ab) | ((bv == ab) & (bi < ai))
        return jnp.where(take_b, bv, av), jnp.where(take_b, bi, ai)

    av, ai = wv[0], wi[0]
    for t in range(1, n_chunk):
        av, ai = comb(av, ai, wv[t], wi[t])
    codes_ref[...] = ai


def _compute_codes(z, e_t, a, c, bm):
    b, d = z.shape
    k = e_t.shape[1]
    return pl.pallas_call(
        _codes_body,
        grid=(b // bm,),
        in_specs=[
            pl.BlockSpec((bm, d), lambda i: (i, 0)),
            pl.BlockSpec((d, k), lambda i: (0, 0)),
            pl.BlockSpec((bm, 1), lambda i: (i, 0)),
            pl.BlockSpec((1, k), lambda i: (0, 0)),
        ],
        out_specs=pl.BlockSpec((bm,), lambda i: (i,)),
        out_shape=jax.ShapeDtypeStruct((b,), jnp.int32),
    )(z, e_t, a, c)


def _make_sc_gather(b, d, k, nw):
    b_per_w = b // nw
    n_chunk = b_per_w // 128
    mesh = plsc.VectorSubcoreMesh(core_axis_name="c", subcore_axis_name="s")

    @functools.partial(
        pl.kernel,
        mesh=mesh,
        out_type=[
            jax.ShapeDtypeStruct((b, d), jnp.float32),
            jax.ShapeDtypeStruct((nw, 16), jnp.float32),
        ],
        scratch_types=[
            pltpu.VMEM((n_chunk, 128), jnp.int32),
            pltpu.VMEM((128, 128), jnp.float32),
            pltpu.VMEM((128, d), jnp.float32),
            pltpu.VMEM((128, d), jnp.float32),
            pltpu.VMEM((16,), jnp.float32),
            pltpu.SemaphoreType.DMA,
        ],
    )
    def sc_kernel(e_hbm, codes_hbm, z_hbm, zq_hbm, part_hbm,
                  idx_v, rows_v, z_v, out_v, acc_v, sem):
        wid = lax.axis_index("s") * 2 + lax.axis_index("c")
        base = wid * b_per_w
        for j in range(n_chunk):
            pltpu.sync_copy(codes_hbm.at[pl.ds(base + j * 128, 128)],
                            idx_v.at[j])

        zero = jnp.zeros((16,), jnp.float32)
        acc0, acc1 = zero, zero
        for j in range(n_chunk):
            cp = pltpu.async_copy(e_hbm.at[idx_v.at[j]], rows_v, sem)
            pltpu.sync_copy(z_hbm.at[pl.ds(base + j * 128, 128)], z_v)
            cp.wait()

            def body(r, acc):
                a0, a1 = acc
                zq0 = rows_v[r, pl.ds(0, 16)]
                z0 = z_v[r, pl.ds(0, 16)]
                d0 = zq0 - z0
                out_v[r, pl.ds(0, 16)] = z0 + d0
                zq1 = rows_v[r, pl.ds(16, 16)]
                z1 = z_v[r, pl.ds(16, 16)]
                d1 = zq1 - z1
                out_v[r, pl.ds(16, 16)] = z1 + d1
                return (a0 + d0 * d0, a1 + d1 * d1)

            acc0, acc1 = lax.fori_loop(0, 128, body, (acc0, acc1))
            pltpu.sync_copy(out_v, zq_hbm.at[pl.ds(base + j * 128, 128)])
        acc_v[...] = acc0 + acc1
        pltpu.sync_copy(acc_v, part_hbm.at[wid])

    return sc_kernel


def kernel(z, embedding_weight):
    b, d = z.shape
    k = embedding_weight.shape[0]
    e_t = embedding_weight.T
    a = jnp.sum(z ** 2, axis=1, keepdims=True)
    c = jnp.sum(embedding_weight ** 2, axis=1).reshape(1, k)
    codes = _compute_codes(-2.0 * z, e_t, a, c, bm=256)
    nw = 32
    e_pad = jnp.pad(embedding_weight, ((0, 0), (0, 128 - d)))
    z_q_st, partials = _make_sc_gather(b, d, k, nw)(
        e_pad, codes, z)
    m = jnp.sum(partials) / (b * d)
    commit_loss = 10.0 * (0.25 * m + m)
    return (z_q_st, commit_loss, codes)


# speedup vs baseline: 1.0559x; 1.0559x over previous
import functools

import jax
import jax.numpy as jnp
from jax import lax
from jax.experimental import pallas as pl
from jax.experimental.pallas import tpu as pltpu
from jax.experimental.pallas import tpu_sc as plsc

_CHUNK = 4096


def _codes_body(zm2_ref, et_ref, a_ref, c_ref, codes_ref):
    zm2 = zm2_ref[...]
    et = et_ref[...]
    k = et.shape[1]
    bm = zm2.shape[0]
    a = a_ref[...]
    c = c_ref[...]
    n_chunk = k // _CHUNK
    cols = _CHUNK // 128
    wv, wi = [], []
    for t in range(n_chunk):
        acc_v = None
        acc_j = None
        for js in range(0, cols, 4):
            lo = t * _CHUNK + js * 128
            m2 = jnp.dot(zm2, et[:, lo:lo + 512],
                         preferred_element_type=jnp.float32)
            vs = (a + m2) + c[:, lo:lo + 512]
            for jj in range(4):
                j = js + jj
                v = vs[:, jj * 128:(jj + 1) * 128]
                if acc_v is None:
                    acc_v = v
                    acc_j = jnp.zeros((bm, 128), jnp.int32)
                else:
                    lt = v < acc_v
                    acc_v = jnp.where(lt, v, acc_v)
                    acc_j = jnp.where(lt, j, acc_j)
        lane = lax.broadcasted_iota(jnp.int32, (bm, 128), 1)
        gk = t * _CHUNK + acc_j * 128 + lane
        mn = jnp.min(acc_v, axis=1, keepdims=True)
        idx = jnp.min(jnp.where(acc_v == mn, gk, k), axis=1)
        wv.append(mn[:, 0])
        wi.append(idx)

    def round_bf16(v):
        u = lax.bitcast_convert_type(v, jnp.uint32)
        lsb = (u >> jnp.uint32(16)) & jnp.uint32(1)
        u = (u + jnp.uint32(0x7FFF) + lsb) & jnp.uint32(0xFFFF0000)
        return lax.bitcast_convert_type(u, jnp.float32)

    def comb(av, ai, bv, bi):
        ab = round_bf16(av)
        take_b = (bv < ab) | ((bv == ab) & (bi < ai))
        return jnp.where(take_b, bv, av), jnp.where(take_b, bi, ai)

    av, ai = wv[0], wi[0]
    for t in range(1, n_chunk):
        av, ai = comb(av, ai, wv[t], wi[t])
    codes_ref[...] = ai


def _compute_codes(z, e_t, a, c, bm):
    b, d = z.shape
    k = e_t.shape[1]
    return pl.pallas_call(
        _codes_body,
        grid=(b // bm,),
        in_specs=[
            pl.BlockSpec((bm, d), lambda i: (i, 0)),
            pl.BlockSpec((d, k), lambda i: (0, 0)),
            pl.BlockSpec((bm, 1), lambda i: (i, 0)),
            pl.BlockSpec((1, k), lambda i: (0, 0)),
        ],
        out_specs=pl.BlockSpec((bm,), lambda i: (i,)),
        out_shape=jax.ShapeDtypeStruct((b,), jnp.int32),
    )(z, e_t, a, c)


def _make_sc_gather(b, d, k, nw):
    b_per_w = b // nw
    n_chunk = b_per_w // 128
    mesh = plsc.VectorSubcoreMesh(core_axis_name="c", subcore_axis_name="s")

    @functools.partial(
        pl.kernel,
        mesh=mesh,
        out_type=[
            jax.ShapeDtypeStruct((b, d), jnp.float32),
            jax.ShapeDtypeStruct((nw, 16), jnp.float32),
        ],
        scratch_types=[
            pltpu.VMEM((n_chunk, 128), jnp.int32),
            pltpu.VMEM((128, 128), jnp.float32),
            pltpu.VMEM((128, d), jnp.float32),
            pltpu.VMEM((128, d), jnp.float32),
            pltpu.VMEM((16,), jnp.float32),
            pltpu.SemaphoreType.DMA,
        ],
    )
    def sc_kernel(e_hbm, codes_hbm, z_hbm, zq_hbm, part_hbm,
                  idx_v, rows_v, z_v, out_v, acc_v, sem):
        wid = lax.axis_index("s") * 2 + lax.axis_index("c")
        base = wid * b_per_w
        for j in range(n_chunk):
            pltpu.sync_copy(codes_hbm.at[pl.ds(base + j * 128, 128)],
                            idx_v.at[j])

        zero = jnp.zeros((16,), jnp.float32)
        acc0, acc1 = zero, zero
        for j in range(n_chunk):
            cp = pltpu.async_copy(e_hbm.at[idx_v.at[j]], rows_v, sem)
            pltpu.sync_copy(z_hbm.at[pl.ds(base + j * 128, 128)], z_v)
            cp.wait()

            def body(r, acc):
                a0, a1 = acc
                zq0 = rows_v[r, pl.ds(0, 16)]
                z0 = z_v[r, pl.ds(0, 16)]
                d0 = zq0 - z0
                out_v[r, pl.ds(0, 16)] = z0 + d0
                zq1 = rows_v[r, pl.ds(16, 16)]
                z1 = z_v[r, pl.ds(16, 16)]
                d1 = zq1 - z1
                out_v[r, pl.ds(16, 16)] = z1 + d1
                return (a0 + d0 * d0, a1 + d1 * d1)

            acc0, acc1 = lax.fori_loop(0, 128, body, (acc0, acc1))
            pltpu.sync_copy(out_v, zq_hbm.at[pl.ds(base + j * 128, 128)])
        acc_v[...] = acc0 + acc1
        pltpu.sync_copy(acc_v, part_hbm.at[wid])

    return sc_kernel


def kernel(z, embedding_weight):
    b, d = z.shape
    k = embedding_weight.shape[0]
    e_t = embedding_weight.T
    a = jnp.sum(z ** 2, axis=1, keepdims=True)
    c = jnp.sum(embedding_weight ** 2, axis=1).reshape(1, k)
    codes = _compute_codes(-2.0 * z, e_t, a, c, bm=256)
    nw = 32
    e_pad = jnp.pad(embedding_weight, ((0, 0), (0, 128 - d)))
    z_q_st, partials = _make_sc_gather(b, d, k, nw)(
        e_pad, codes, z)
    m = jnp.sum(partials) / (b * d)
    commit_loss = 10.0 * (0.25 * m + m)
    return (z_q_st, commit_loss, codes)


# speedup vs baseline: 1.0965x; 1.0384x over previous
import functools

import jax
import jax.numpy as jnp
from jax import lax
from jax.experimental import pallas as pl
from jax.experimental.pallas import tpu as pltpu
from jax.experimental.pallas import tpu_sc as plsc

_CHUNK = 4096


def _codes_body(zm2_ref, et_ref, a_ref, c_ref, codes_ref):
    zm2 = zm2_ref[...]
    et = et_ref[...]
    k = et.shape[1]
    bm = zm2.shape[0]
    a = a_ref[...]
    c = c_ref[...]
    n_chunk = k // _CHUNK
    cols = _CHUNK // 128
    wv, wi = [], []
    for t in range(n_chunk):
        acc_v = None
        acc_j = None
        for js in range(0, cols, 4):
            lo = t * _CHUNK + js * 128
            m2 = jnp.dot(zm2, et[:, lo:lo + 512],
                         preferred_element_type=jnp.float32)
            vs = (a + m2) + c[:, lo:lo + 512]
            for jj in range(4):
                j = js + jj
                v = vs[:, jj * 128:(jj + 1) * 128]
                if acc_v is None:
                    acc_v = v
                    acc_j = jnp.zeros((bm, 128), jnp.int32)
                else:
                    lt = v < acc_v
                    acc_v = jnp.where(lt, v, acc_v)
                    acc_j = jnp.where(lt, j, acc_j)
        lane = lax.broadcasted_iota(jnp.int32, (bm, 128), 1)
        gk = t * _CHUNK + acc_j * 128 + lane
        mn = jnp.min(acc_v, axis=1, keepdims=True)
        idx = jnp.min(jnp.where(acc_v == mn, gk, k), axis=1)
        wv.append(mn[:, 0])
        wi.append(idx)

    def round_bf16(v):
        u = lax.bitcast_convert_type(v, jnp.uint32)
        lsb = (u >> jnp.uint32(16)) & jnp.uint32(1)
        u = (u + jnp.uint32(0x7FFF) + lsb) & jnp.uint32(0xFFFF0000)
        return lax.bitcast_convert_type(u, jnp.float32)

    def comb(av, ai, bv, bi):
        ab = round_bf16(av)
        take_b = (bv < ab) | ((bv == ab) & (bi < ai))
        return jnp.where(take_b, bv, av), jnp.where(take_b, bi, ai)

    av, ai = wv[0], wi[0]
    for t in range(1, n_chunk):
        av, ai = comb(av, ai, wv[t], wi[t])
    codes_ref[...] = ai


def _compute_codes(z, e_t, a, c, bm):
    b, d = z.shape
    k = e_t.shape[1]
    return pl.pallas_call(
        _codes_body,
        grid=(b // bm,),
        in_specs=[
            pl.BlockSpec((bm, d), lambda i: (i, 0)),
            pl.BlockSpec((d, k), lambda i: (0, 0)),
            pl.BlockSpec((bm, 1), lambda i: (i, 0)),
            pl.BlockSpec((1, k), lambda i: (0, 0)),
        ],
        out_specs=pl.BlockSpec((bm,), lambda i: (i,)),
        out_shape=jax.ShapeDtypeStruct((b,), jnp.int32),
    )(z, e_t, a, c)


def _make_sc_gather(b, d, k, nw):
    b_per_w = b // nw
    n_chunk = b_per_w // 128
    mesh = plsc.VectorSubcoreMesh(core_axis_name="c", subcore_axis_name="s")

    @functools.partial(
        pl.kernel,
        mesh=mesh,
        out_type=[
            jax.ShapeDtypeStruct((b, d), jnp.float32),
            jax.ShapeDtypeStruct((nw, 16), jnp.float32),
        ],
        scratch_types=[
            pltpu.VMEM((n_chunk, 128), jnp.int32),
            pltpu.VMEM((128, 128), jnp.float32),
            pltpu.VMEM((128, d), jnp.float32),
            pltpu.VMEM((128, d), jnp.float32),
            pltpu.VMEM((16,), jnp.float32),
            pltpu.SemaphoreType.DMA,
        ],
    )
    def sc_kernel(e_hbm, codes_hbm, z_hbm, zq_hbm, part_hbm,
                  idx_v, rows_v, z_v, out_v, acc_v, sem):
        wid = lax.axis_index("s") * 2 + lax.axis_index("c")
        base = wid * b_per_w
        for j in range(n_chunk):
            pltpu.sync_copy(codes_hbm.at[pl.ds(base + j * 128, 128)],
                            idx_v.at[j])

        zero = jnp.zeros((16,), jnp.float32)
        acc0, acc1 = zero, zero
        for j in range(n_chunk):
            cp = pltpu.async_copy(e_hbm.at[idx_v.at[j]], rows_v, sem)
            pltpu.sync_copy(z_hbm.at[pl.ds(base + j * 128, 128)], z_v)
            cp.wait()

            def body(r, acc):
                a0, a1 = acc
                zq0 = rows_v[r, pl.ds(0, 16)]
                z0 = z_v[r, pl.ds(0, 16)]
                d0 = zq0 - z0
                out_v[r, pl.ds(0, 16)] = z0 + d0
                zq1 = rows_v[r, pl.ds(16, 16)]
                z1 = z_v[r, pl.ds(16, 16)]
                d1 = zq1 - z1
                out_v[r, pl.ds(16, 16)] = z1 + d1
                return (a0 + d0 * d0, a1 + d1 * d1)

            acc0, acc1 = lax.fori_loop(0, 128, body, (acc0, acc1))
            pltpu.sync_copy(out_v, zq_hbm.at[pl.ds(base + j * 128, 128)])
        acc_v[...] = acc0 + acc1
        pltpu.sync_copy(acc_v, part_hbm.at[wid])

    return sc_kernel


def kernel(z, embedding_weight):
    b, d = z.shape
    k = embedding_weight.shape[0]
    e_t = embedding_weight.T
    a = jnp.sum(z ** 2, axis=1, keepdims=True)
    c = jnp.sum(embedding_weight ** 2, axis=1).reshape(1, k)
    codes = _compute_codes(-2.0 * z, e_t, a, c, bm=512)
    nw = 32
    e_pad = jnp.pad(embedding_weight, ((0, 0), (0, 128 - d)))
    z_q_st, partials = _make_sc_gather(b, d, k, nw)(
        e_pad, codes, z)
    m = jnp.sum(partials) / (b * d)
    commit_loss = 10.0 * (0.25 * m + m)
    return (z_q_st, commit_loss, codes)


# speedup vs baseline: 1.1108x; 1.0131x over previous
import functools

import jax
import jax.numpy as jnp
from jax import lax
from jax.experimental import pallas as pl
from jax.experimental.pallas import tpu as pltpu
from jax.experimental.pallas import tpu_sc as plsc

_CHUNK = 4096


def _codes_body(zm2_ref, et_ref, a_ref, c_ref, codes_ref):
    zm2 = zm2_ref[...]
    et = et_ref[...]
    k = et.shape[1]
    bm = zm2.shape[0]
    a = a_ref[...]
    c = c_ref[...]
    n_chunk = k // _CHUNK
    cols = _CHUNK // 128
    wv, wi = [], []
    for t in range(n_chunk):
        acc_v = None
        acc_j = None
        for js in range(0, cols, 4):
            lo = t * _CHUNK + js * 128
            m2 = jnp.dot(zm2, et[:, lo:lo + 512],
                         preferred_element_type=jnp.float32)
            vs = (a + m2) + c[:, lo:lo + 512]
            for jj in range(4):
                j = js + jj
                v = vs[:, jj * 128:(jj + 1) * 128]
                if acc_v is None:
                    acc_v = v
                    acc_j = jnp.zeros((bm, 128), jnp.int32)
                else:
                    lt = v < acc_v
                    acc_v = jnp.where(lt, v, acc_v)
                    acc_j = jnp.where(lt, j, acc_j)
        lane = lax.broadcasted_iota(jnp.int32, (bm, 128), 1)
        gk = t * _CHUNK + acc_j * 128 + lane
        mn = jnp.min(acc_v, axis=1, keepdims=True)
        idx = jnp.min(jnp.where(acc_v == mn, gk, k), axis=1)
        wv.append(mn[:, 0])
        wi.append(idx)

    def round_bf16(v):
        u = lax.bitcast_convert_type(v, jnp.uint32)
        lsb = (u >> jnp.uint32(16)) & jnp.uint32(1)
        u = (u + jnp.uint32(0x7FFF) + lsb) & jnp.uint32(0xFFFF0000)
        return lax.bitcast_convert_type(u, jnp.float32)

    def comb(av, ai, bv, bi):
        ab = round_bf16(av)
        take_b = (bv < ab) | ((bv == ab) & (bi < ai))
        return jnp.where(take_b, bv, av), jnp.where(take_b, bi, ai)

    av, ai = wv[0], wi[0]
    for t in range(1, n_chunk):
        av, ai = comb(av, ai, wv[t], wi[t])
    codes_ref[...] = ai


def _compute_codes(z, e_t, a, c, bm):
    b, d = z.shape
    k = e_t.shape[1]
    return pl.pallas_call(
        _codes_body,
        grid=(b // bm,),
        in_specs=[
            pl.BlockSpec((bm, d), lambda i: (i, 0)),
            pl.BlockSpec((d, k), lambda i: (0, 0)),
            pl.BlockSpec((bm, 1), lambda i: (i, 0)),
            pl.BlockSpec((1, k), lambda i: (0, 0)),
        ],
        out_specs=pl.BlockSpec((bm,), lambda i: (i,)),
        out_shape=jax.ShapeDtypeStruct((b,), jnp.int32),
    )(z, e_t, a, c)


def _make_sc_gather(b, d, k, nw):
    b_per_w = b // nw
    n_chunk = b_per_w // 128
    mesh = plsc.VectorSubcoreMesh(core_axis_name="c", subcore_axis_name="s")

    @functools.partial(
        pl.kernel,
        mesh=mesh,
        out_type=[
            jax.ShapeDtypeStruct((b, d), jnp.float32),
            jax.ShapeDtypeStruct((nw, 16), jnp.float32),
        ],
        scratch_types=[
            pltpu.VMEM((n_chunk, 128), jnp.int32),
            pltpu.VMEM((128, 128), jnp.float32),
            pltpu.VMEM((128, d), jnp.float32),
            pltpu.VMEM((128, d), jnp.float32),
            pltpu.VMEM((16,), jnp.float32),
            pltpu.SemaphoreType.DMA,
        ],
    )
    def sc_kernel(e_hbm, codes_hbm, z_hbm, zq_hbm, part_hbm,
                  idx_v, rows_v, z_v, out_v, acc_v, sem):
        wid = lax.axis_index("s") * 2 + lax.axis_index("c")
        base = wid * b_per_w
        for j in range(n_chunk):
            pltpu.sync_copy(codes_hbm.at[pl.ds(base + j * 128, 128)],
                            idx_v.at[j])

        zero = jnp.zeros((16,), jnp.float32)
        acc0, acc1 = zero, zero
        for j in range(n_chunk):
            cp = pltpu.async_copy(e_hbm.at[idx_v.at[j]], rows_v, sem)
            pltpu.sync_copy(z_hbm.at[pl.ds(base + j * 128, 128)], z_v)
            cp.wait()

            def body(r, acc):
                a0, a1 = acc
                zq0 = rows_v[r, pl.ds(0, 16)]
                z0 = z_v[r, pl.ds(0, 16)]
                d0 = zq0 - z0
                out_v[r, pl.ds(0, 16)] = z0 + d0
                zq1 = rows_v[r, pl.ds(16, 16)]
                z1 = z_v[r, pl.ds(16, 16)]
                d1 = zq1 - z1
                out_v[r, pl.ds(16, 16)] = z1 + d1
                return (a0 + d0 * d0, a1 + d1 * d1)

            acc0, acc1 = lax.fori_loop(0, 128, body, (acc0, acc1))
            pltpu.sync_copy(out_v, zq_hbm.at[pl.ds(base + j * 128, 128)])
        acc_v[...] = acc0 + acc1
        pltpu.sync_copy(acc_v, part_hbm.at[wid])

    return sc_kernel


def kernel(z, embedding_weight):
    b, d = z.shape
    k = embedding_weight.shape[0]
    e_t = embedding_weight.T
    a = jnp.sum(z ** 2, axis=1, keepdims=True)
    c = jnp.sum(embedding_weight ** 2, axis=1).reshape(1, k)
    codes = _compute_codes(-2.0 * z, e_t, a, c, bm=1024)
    nw = 32
    e_pad = jnp.pad(embedding_weight, ((0, 0), (0, 128 - d)))
    z_q_st, partials = _make_sc_gather(b, d, k, nw)(
        e_pad, codes, z)
    m = jnp.sum(partials) / (b * d)
    commit_loss = 10.0 * (0.25 * m + m)
    return (z_q_st, commit_loss, codes)


# speedup vs baseline: 1.1481x; 1.0335x over previous
import functools

import jax
import jax.numpy as jnp
from jax import lax
from jax.experimental import pallas as pl
from jax.experimental.pallas import tpu as pltpu
from jax.experimental.pallas import tpu_sc as plsc

_CHUNK = 4096


def _codes_body(zm2_ref, et_ref, a_ref, c_ref, codes_ref):
    zm2 = zm2_ref[...]
    et = et_ref[...]
    k = et.shape[1]
    bm = zm2.shape[0]
    a = a_ref[...]
    c = c_ref[...]
    n_chunk = k // _CHUNK
    cols = _CHUNK // 128
    wv, wi = [], []
    for t in range(n_chunk):
        acc_v = None
        acc_j = None
        for js in range(0, cols, 4):
            lo = t * _CHUNK + js * 128
            m2 = jnp.dot(zm2, et[:, lo:lo + 512],
                         preferred_element_type=jnp.float32)
            vs = (a + m2) + c[:, lo:lo + 512]
            for jj in range(4):
                j = js + jj
                v = vs[:, jj * 128:(jj + 1) * 128]
                if acc_v is None:
                    acc_v = v
                    acc_j = jnp.zeros((bm, 128), jnp.int32)
                else:
                    lt = v < acc_v
                    acc_v = jnp.where(lt, v, acc_v)
                    acc_j = jnp.where(lt, j, acc_j)
        lane = lax.broadcasted_iota(jnp.int32, (bm, 128), 1)
        gk = t * _CHUNK + acc_j * 128 + lane
        mn = jnp.min(acc_v, axis=1, keepdims=True)
        idx = jnp.min(jnp.where(acc_v == mn, gk, k), axis=1)
        wv.append(mn[:, 0])
        wi.append(idx)

    def round_bf16(v):
        u = lax.bitcast_convert_type(v, jnp.uint32)
        lsb = (u >> jnp.uint32(16)) & jnp.uint32(1)
        u = (u + jnp.uint32(0x7FFF) + lsb) & jnp.uint32(0xFFFF0000)
        return lax.bitcast_convert_type(u, jnp.float32)

    def comb(av, ai, bv, bi):
        ab = round_bf16(av)
        take_b = (bv < ab) | ((bv == ab) & (bi < ai))
        return jnp.where(take_b, bv, av), jnp.where(take_b, bi, ai)

    av, ai = wv[0], wi[0]
    for t in range(1, n_chunk):
        av, ai = comb(av, ai, wv[t], wi[t])
    codes_ref[...] = ai


def _compute_codes(z, e_t, a, c, bm):
    b, d = z.shape
    k = e_t.shape[1]
    return pl.pallas_call(
        _codes_body,
        grid=(b // bm,),
        in_specs=[
            pl.BlockSpec((bm, d), lambda i: (i, 0)),
            pl.BlockSpec((d, k), lambda i: (0, 0)),
            pl.BlockSpec((bm, 1), lambda i: (i, 0)),
            pl.BlockSpec((1, k), lambda i: (0, 0)),
        ],
        out_specs=pl.BlockSpec((bm,), lambda i: (i,)),
        out_shape=jax.ShapeDtypeStruct((b,), jnp.int32),
    )(z, e_t, a, c)


def _make_sc_gather(b, d, k, nw):
    b_per_w = b // nw
    n_chunk = b_per_w // 128
    mesh = plsc.VectorSubcoreMesh(core_axis_name="c", subcore_axis_name="s")

    @functools.partial(
        pl.kernel,
        mesh=mesh,
        out_type=[
            jax.ShapeDtypeStruct((b, d), jnp.float32),
            jax.ShapeDtypeStruct((nw, 16), jnp.float32),
        ],
        scratch_types=[
            pltpu.VMEM((n_chunk, 128), jnp.int32),
            pltpu.VMEM((128, 128), jnp.float32),
            pltpu.VMEM((128, d), jnp.float32),
            pltpu.VMEM((128, d), jnp.float32),
            pltpu.VMEM((16,), jnp.float32),
            pltpu.SemaphoreType.DMA,
        ],
    )
    def sc_kernel(e_hbm, codes_hbm, z_hbm, zq_hbm, part_hbm,
                  idx_v, rows_v, z_v, out_v, acc_v, sem):
        wid = lax.axis_index("s") * 2 + lax.axis_index("c")
        base = wid * b_per_w
        for j in range(n_chunk):
            pltpu.sync_copy(codes_hbm.at[pl.ds(base + j * 128, 128)],
                            idx_v.at[j])

        zero = jnp.zeros((16,), jnp.float32)
        acc0, acc1 = zero, zero
        for j in range(n_chunk):
            cp = pltpu.async_copy(e_hbm.at[idx_v.at[j]], rows_v, sem)
            pltpu.sync_copy(z_hbm.at[pl.ds(base + j * 128, 128)], z_v)
            cp.wait()

            def body(r, acc):
                a0, a1 = acc
                zq0 = rows_v[r, pl.ds(0, 16)]
                z0 = z_v[r, pl.ds(0, 16)]
                d0 = zq0 - z0
                out_v[r, pl.ds(0, 16)] = z0 + d0
                zq1 = rows_v[r, pl.ds(16, 16)]
                z1 = z_v[r, pl.ds(16, 16)]
                d1 = zq1 - z1
                out_v[r, pl.ds(16, 16)] = z1 + d1
                return (a0 + d0 * d0, a1 + d1 * d1)

            acc0, acc1 = lax.fori_loop(0, 128, body, (acc0, acc1))
            pltpu.sync_copy(out_v, zq_hbm.at[pl.ds(base + j * 128, 128)])
        acc_v[...] = acc0 + acc1
        pltpu.sync_copy(acc_v, part_hbm.at[wid])

    return sc_kernel


def kernel(z, embedding_weight):
    b, d = z.shape
    k = embedding_weight.shape[0]
    e_t = embedding_weight.T
    a = jnp.sum(z ** 2, axis=1, keepdims=True)
    c = jnp.sum(embedding_weight ** 2, axis=1).reshape(1, k)
    codes = _compute_codes(-2.0 * z, e_t, a, c, bm=2048)
    nw = 32
    e_pad = jnp.pad(embedding_weight, ((0, 0), (0, 128 - d)))
    z_q_st, partials = _make_sc_gather(b, d, k, nw)(
        e_pad, codes, z)
    m = jnp.sum(partials) / (b * d)
    commit_loss = 10.0 * (0.25 * m + m)
    return (z_q_st, commit_loss, codes)
